# baseline (speedup 1.0000x reference)
import functools

import jax
import jax.numpy as jnp
from jax import lax
from jax.experimental import pallas as pl
from jax.experimental.pallas import tpu as pltpu
from jax.experimental.pallas import tpu_sc as plsc

C_Z = 128
N_BINS = 64
MIN_D = 0.0
MAX_D = 50.0
N = 512
NTOT = N * N

BIN_W = MAX_D / N_BINS
INV_W = N_BINS / MAX_D
CLIP_HI = MAX_D - 1e-6

NC, NS = 2, 16
NW = NC * NS
ROWS_PER_TILE = NTOT // NW
CHUNK = 256
NCHUNK = ROWS_PER_TILE // CHUNK


def _table_body(we_ref, lnw_ref, lnb_ref, wp_ref, out_ref):
    we = we_ref[...]
    mu = jnp.mean(we, axis=1, keepdims=True)
    var = jnp.mean((we - mu) ** 2, axis=1, keepdims=True)
    x = (we - mu) / jnp.sqrt(var + 1e-5) * lnw_ref[...] + lnb_ref[...]
    out_ref[...] = lax.dot_general(x, wp_ref[...], (((1,), (1,)), ((), ())),
                                   preferred_element_type=jnp.float32)


_table_call = pl.pallas_call(
    _table_body, out_shape=jax.ShapeDtypeStruct((N_BINS, C_Z), jnp.float32))


def _bin16(d):
    d = jnp.minimum(jnp.maximum(d, MIN_D), CLIP_HI)
    c0 = jnp.clip((d * INV_W).astype(jnp.int32), 0, N_BINS - 1)
    e0 = c0.astype(jnp.float32) * BIN_W
    e1 = (c0 + 1).astype(jnp.float32) * BIN_W
    k = jnp.where(d <= e0, c0 - 1, jnp.where(d > e1, c0 + 1, c0))
    return jnp.clip(k, 0, N_BINS - 1)


SUB = 128
NSUB = ROWS_PER_TILE // SUB
SLOTS = 6


@functools.cache
def _make_sc_gather():
    scratch = [
        pltpu.VMEM((ROWS_PER_TILE,), jnp.float32),
        pltpu.VMEM((ROWS_PER_TILE,), jnp.int32),
        pltpu.VMEM_SHARED((N_BINS, C_Z), jnp.float32),
        pltpu.VMEM((SLOTS, SUB, C_Z), jnp.float32),
        pltpu.SemaphoreType.DMA((SLOTS,)),
        pltpu.SemaphoreType.DMA((SLOTS,)),
    ]

    @functools.partial(
        pl.kernel,
        mesh=plsc.VectorSubcoreMesh(core_axis_name="c", subcore_axis_name="s"),
        out_type=jax.ShapeDtypeStruct((NTOT, C_Z), jnp.float32),
        scratch_types=scratch,
        compiler_params=pltpu.CompilerParams(needs_layout_passes=False),
    )
    def _sc_gather(d_hbm, table_hbm, out_hbm, d_v, idx_v, table_sh,
                   stage, gsem, wsem):
        sid = lax.axis_index("s")
        wid = sid * NC + lax.axis_index("c")
        base = wid * ROWS_PER_TILE

        @pl.when(sid == 0)
        def _():
            pltpu.sync_copy(table_hbm, table_sh)

        pltpu.sync_copy(d_hbm.at[pl.ds(base, ROWS_PER_TILE)], d_v)

        def compute_idx(sub):
            @plsc.parallel_loop(0, SUB // 16, unroll=1)
            def idx_body(i):
                off = sub * SUB + i * 16
                idx_v[pl.ds(off, 16)] = _bin16(d_v[pl.ds(off, 16)])

        for b in range(SLOTS):
            compute_idx(b)

        plsc.subcore_barrier()

        def s_gather(sid_, b):
            idx_slice = idx_v.at[pl.ds(sid_ * SUB, SUB)]
            return pltpu.make_async_copy(table_sh.at[idx_slice],
                                         stage.at[b], gsem.at[b])

        def s_copy(sid_, b):
            dst = out_hbm.at[pl.ds(base + sid_ * SUB, SUB)]
            return pltpu.make_async_copy(stage.at[b], dst, wsem.at[b])

        for b in range(SLOTS):
            s_gather(b, b).start()

        LAG = 2

        def sub_body(rnd, carry):
            rr = lax.rem(rnd, SLOTS)
            s_gather(rnd, rr).wait()
            s_copy(rnd, rr).start()

            @pl.when(rnd + SLOTS < NSUB)
            def _():
                compute_idx(rnd + SLOTS)

            j = rnd - LAG
            jr = lax.rem(rnd + SLOTS - LAG, SLOTS)

            @pl.when(jnp.logical_and(j >= 0, j + SLOTS < NSUB))
            def _():
                s_copy(j, jr).wait()
                s_gather(j + SLOTS, jr).start()

            return carry

        lax.fori_loop(0, NSUB, sub_body, 0)
        for k in range(NSUB - SLOTS, NSUB):
            s_copy(k, k % SLOTS).wait()

    return _sc_gather


def kernel(distance_constraints, W_embed, ln_weight, ln_bias, W_proj):
    table = _table_call(W_embed.T, ln_weight.reshape(1, C_Z),
                        ln_bias.reshape(1, C_Z), W_proj)
    d_flat = distance_constraints.reshape(NTOT)
    out = _make_sc_gather()(d_flat, table)
    return out.reshape(1, N, N, C_Z)
